# baseline (speedup 1.0000x reference)
import jax
import jax.numpy as jnp
from jax import lax
from jax.experimental import pallas as pl
from jax.experimental.pallas import tpu as pltpu

WT, WH, WW = 16, 8, 8
NHEADS = 32
NT = 2 * WT - 1
NHW = (2 * WH - 1) * (2 * WW - 1)
Q = WH * WW
QQ = Q * Q
HG = 16


def _build_g_body(t_ref, d_ref, o_ref):
    oh = (lax.broadcasted_iota(jnp.int32, (NHW, QQ), 0) == d_ref[...]).astype(
        jnp.float32
    )
    o_ref[...] = jnp.dot(t_ref[...], oh, preferred_element_type=jnp.float32)


def _gcat_body(ga_ref, gb_ref, gc_ref, oe_ref, oo_ref):
    oe_ref[:, :, 0:Q] = ga_ref[:, 0]
    oe_ref[:, :, Q : 2 * Q] = gb_ref[:, 0]
    oo_ref[:, :, 0:Q] = gb_ref[:, 0]
    oo_ref[:, :, Q : 2 * Q] = gc_ref[:, 0]


def _copy_body(ge_ref, go_ref, o_ref, sem):
    i = pl.program_id(1)
    ke = ((WT - 1 - i) // 2) * (2 * Q)
    ko = ((WT - 2 - i) // 2) * (2 * Q)

    @pl.when(i % 2 == 1)
    def _():
        cp = pltpu.make_async_copy(
            ge_ref.at[:, :, pl.ds(ke, WT * Q)], o_ref.at[:, 0], sem
        )
        cp.start()
        cp.wait()

    @pl.when(i % 2 == 0)
    def _():
        cp = pltpu.make_async_copy(
            go_ref.at[:, :, pl.ds(ko, WT * Q)], o_ref.at[:, 0], sem
        )
        cp.start()
        cp.wait()


def kernel(relative_position_bias_table, rel_index):
    table = relative_position_bias_table
    r4 = rel_index.reshape(WT, Q, WT, Q)
    dhw = r4[0, :, WT - 1, :].reshape(1, QQ)

    tableT = (
        table.reshape(NT, NHW, NHEADS).transpose(2, 0, 1).reshape(NHEADS * NT, NHW)
    )

    g = pl.pallas_call(
        _build_g_body,
        in_specs=[
            pl.BlockSpec((NHEADS * NT, NHW), lambda: (0, 0)),
            pl.BlockSpec((1, QQ), lambda: (0, 0)),
        ],
        out_specs=pl.BlockSpec((NHEADS * NT, QQ), lambda: (0, 0)),
        out_shape=jax.ShapeDtypeStruct((NHEADS * NT, QQ), jnp.float32),
    )(tableT, dhw)

    g4 = g.reshape(NHEADS, NT, Q, Q)

    gcat_e, gcat_o = pl.pallas_call(
        _gcat_body,
        grid=(WT,),
        in_specs=[
            pl.BlockSpec((NHEADS, 1, Q, Q), lambda p: (0, 2 * WT - 2 - 2 * p, 0, 0)),
            pl.BlockSpec(
                (NHEADS, 1, Q, Q),
                lambda p: (0, jnp.maximum(2 * WT - 3 - 2 * p, 0), 0, 0),
            ),
            pl.BlockSpec(
                (NHEADS, 1, Q, Q),
                lambda p: (0, jnp.maximum(2 * WT - 4 - 2 * p, 0), 0, 0),
            ),
        ],
        out_specs=[
            pl.BlockSpec((NHEADS, Q, 2 * Q), lambda p: (0, 0, p)),
            pl.BlockSpec((NHEADS, Q, 2 * Q), lambda p: (0, 0, p)),
        ],
        out_shape=[
            jax.ShapeDtypeStruct((NHEADS, Q, 2 * WT * Q), jnp.float32),
            jax.ShapeDtypeStruct((NHEADS, Q, 2 * WT * Q), jnp.float32),
        ],
    )(g4, g4, g4)

    out4 = pl.pallas_call(
        _copy_body,
        grid=(NHEADS // HG, WT),
        in_specs=[
            pl.BlockSpec((HG, Q, 2 * WT * Q), lambda h, i: (h, 0, 0)),
            pl.BlockSpec((HG, Q, 2 * WT * Q), lambda h, i: (h, 0, 0)),
        ],
        out_specs=pl.BlockSpec((HG, 1, Q, WT * Q), lambda h, i: (h, i, 0, 0)),
        out_shape=jax.ShapeDtypeStruct((NHEADS, WT, Q, WT * Q), jnp.float32),
        scratch_shapes=[pltpu.SemaphoreType.DMA],
    )(gcat_e, gcat_o)
    return out4.reshape(NHEADS, WT * Q, WT * Q)
